# Initial kernel scaffold; baseline (speedup 1.0000x reference)
import jax
import jax.numpy as jnp
from jax.experimental import pallas as pl


def kernel(latents, embedding_weight):
    raise NotImplementedError("write your pallas kernel here")


# speedup vs baseline: 1.4372x; 1.4372x over previous
import functools

import jax
import jax.numpy as jnp
from jax import lax
from jax.experimental import pallas as pl
from jax.experimental.pallas import tpu as pltpu
from jax.experimental.pallas import tpu_sc as plsc

_NUM_EMB = 1024
_DIM = 64
_CC = 0.25
_ROWS = 16384
_TILE = 1024
_GRID = _ROWS // _TILE


def _vq_tc_body(z_ref, e_ref, idx_ref, loss_ref, perp_ref, counts_ref,
                sse_ref):
    i = pl.program_id(0)
    z = z_ref[...]
    e = e_ref[...]
    mm = lax.dot_general(z, e, (((1,), (1,)), ((), ())))
    z2 = jnp.sum(z * z, axis=1, keepdims=True)
    e2 = jnp.sum(e * e, axis=1)
    s = z2 + e2[None, :] - 2.0 * mm
    md = jnp.min(s, axis=1)
    cols = lax.broadcasted_iota(jnp.int32, (_TILE, _NUM_EMB), 1)
    idx = jnp.min(jnp.where(s == md[:, None], cols, _NUM_EMB), axis=1)
    idx_ref[...] = idx
    onehot = (idx[:, None] == lax.broadcasted_iota(
        jnp.int32, (_TILE, _NUM_EMB), 1)).astype(jnp.float32)
    cb = jnp.sum(onehot, axis=0)

    @pl.when(i == 0)
    def _():
        counts_ref[...] = cb
        sse_ref[0] = jnp.sum(md)

    @pl.when(i > 0)
    def _():
        counts_ref[...] = counts_ref[...] + cb
        sse_ref[0] = sse_ref[0] + jnp.sum(md)

    @pl.when(i == _GRID - 1)
    def _():
        loss_ref[0, 0] = (1.0 + _CC) * sse_ref[0] / (_ROWS * _DIM)
        p = counts_ref[...] * (1.0 / _ROWS)
        ent = jnp.sum(p * jnp.log(p + 1e-10))
        perp_ref[0, 0] = jnp.exp(-ent)


def _tc_call(flat, emb):
    return pl.pallas_call(
        _vq_tc_body,
        grid=(_GRID,),
        in_specs=[
            pl.BlockSpec((_TILE, _DIM), lambda i: (i, 0)),
            pl.BlockSpec((_NUM_EMB, _DIM), lambda i: (0, 0)),
        ],
        out_specs=[
            pl.BlockSpec((_TILE,), lambda i: (i,)),
            pl.BlockSpec(memory_space=pltpu.SMEM),
            pl.BlockSpec(memory_space=pltpu.SMEM),
        ],
        out_shape=[
            jax.ShapeDtypeStruct((_ROWS,), jnp.int32),
            jax.ShapeDtypeStruct((1, 1), jnp.float32),
            jax.ShapeDtypeStruct((1, 1), jnp.float32),
        ],
        scratch_shapes=[
            pltpu.VMEM((_NUM_EMB,), jnp.float32),
            pltpu.SMEM((1,), jnp.float32),
        ],
    )(flat, emb)


_NC, _NS = 2, 16
_NW = _NC * _NS
_BPW = _ROWS // _NW


_DPAD = 128


@functools.cache
def _sc_gather_call():
    @functools.partial(
        pl.kernel,
        mesh=plsc.VectorSubcoreMesh(core_axis_name="c", subcore_axis_name="s"),
        out_type=jax.ShapeDtypeStruct((_ROWS, _DPAD), jnp.float32),
        scratch_types=[
            pltpu.VMEM((_BPW,), jnp.int32),
            pltpu.VMEM((_BPW, _DPAD), jnp.float32),
            pltpu.SemaphoreType.DMA,
        ],
    )
    def _sc_gather(table_hbm, idx_hbm, out_hbm, idx_v, rows_v, sem):
        wid = lax.axis_index("s") * _NC + lax.axis_index("c")
        base = wid * _BPW
        pltpu.sync_copy(idx_hbm.at[pl.ds(base, _BPW)], idx_v)
        pltpu.async_copy(table_hbm.at[idx_v], rows_v, sem).wait()
        pltpu.sync_copy(rows_v, out_hbm.at[pl.ds(base, _BPW)])

    return _sc_gather


def kernel(latents, embedding_weight):
    b, d, h, w = latents.shape
    flat = jnp.transpose(latents, (0, 2, 3, 1)).reshape(b * h * w, d)
    idx, loss, perp = _tc_call(flat, embedding_weight)
    emb_pad = jnp.pad(embedding_weight, ((0, 0), (0, _DPAD - d)))
    qflat = _sc_gather_call()(emb_pad, idx)[:, :d]
    quantized = jnp.transpose(qflat.reshape(b, h, w, d), (0, 3, 1, 2))
    return quantized, loss[0, 0], perp[0, 0]


# speedup vs baseline: 1.5119x; 1.0519x over previous
import functools

import jax
import jax.numpy as jnp
from jax import lax
from jax.experimental import pallas as pl
from jax.experimental.pallas import tpu as pltpu
from jax.experimental.pallas import tpu_sc as plsc

_NUM_EMB = 1024
_DIM = 64
_CC = 0.25
_ROWS = 16384
_TILE = 1024
_GRID = _ROWS // _TILE


def _vq_tc_body(z_ref, e_ref, idx_ref, loss_ref, perp_ref, counts_ref,
                sse_ref):
    i = pl.program_id(0)
    zt = z_ref[0]
    e = e_ref[...]
    mmn = lax.dot_general(e, -2.0 * zt, (((1,), (0,)), ((), ())))
    z2 = jnp.sum(zt * zt, axis=0)
    e2 = jnp.sum(e * e, axis=1)
    s = (e2[:, None] + z2[None, :]) + mmn
    md = jnp.min(s, axis=0)
    rows = lax.broadcasted_iota(jnp.int32, (_NUM_EMB, _TILE), 0)
    idx = jnp.min(jnp.where(s == md[None, :], rows, _NUM_EMB), axis=0)
    idx_ref[...] = idx
    onehot = (rows == idx[None, :]).astype(jnp.float32)
    cb = lax.dot_general(onehot, jnp.ones((_TILE, 1), jnp.float32),
                         (((1,), (0,)), ((), ())))

    @pl.when(i == 0)
    def _():
        counts_ref[...] = cb
        sse_ref[0] = jnp.sum(md)

    @pl.when(i > 0)
    def _():
        counts_ref[...] = counts_ref[...] + cb
        sse_ref[0] = sse_ref[0] + jnp.sum(md)

    @pl.when(i == _GRID - 1)
    def _():
        loss_ref[0, 0] = (1.0 + _CC) * sse_ref[0] / (_ROWS * _DIM)
        p = counts_ref[...] * (1.0 / _ROWS)
        ent = jnp.sum(p * jnp.log(p + 1e-10))
        perp_ref[0, 0] = jnp.exp(-ent)


def _tc_call(zt3, emb):
    return pl.pallas_call(
        _vq_tc_body,
        grid=(_GRID,),
        in_specs=[
            pl.BlockSpec((1, _DIM, _TILE), lambda i: (i, 0, 0)),
            pl.BlockSpec((_NUM_EMB, _DIM), lambda i: (0, 0)),
        ],
        out_specs=[
            pl.BlockSpec((_TILE,), lambda i: (i,)),
            pl.BlockSpec(memory_space=pltpu.SMEM),
            pl.BlockSpec(memory_space=pltpu.SMEM),
        ],
        out_shape=[
            jax.ShapeDtypeStruct((_ROWS,), jnp.int32),
            jax.ShapeDtypeStruct((1, 1), jnp.float32),
            jax.ShapeDtypeStruct((1, 1), jnp.float32),
        ],
        scratch_shapes=[
            pltpu.VMEM((_NUM_EMB, 1), jnp.float32),
            pltpu.SMEM((1,), jnp.float32),
        ],
    )(zt3, emb)


_NC, _NS = 2, 16
_NW = _NC * _NS
_BPW = _ROWS // _NW


_DPAD = 128


@functools.cache
def _sc_gather_call():
    @functools.partial(
        pl.kernel,
        mesh=plsc.VectorSubcoreMesh(core_axis_name="c", subcore_axis_name="s"),
        out_type=jax.ShapeDtypeStruct((_ROWS, _DPAD), jnp.float32),
        scratch_types=[
            pltpu.VMEM((_BPW,), jnp.int32),
            pltpu.VMEM((_BPW, _DPAD), jnp.float32),
            pltpu.SemaphoreType.DMA,
        ],
    )
    def _sc_gather(table_hbm, idx_hbm, out_hbm, idx_v, rows_v, sem):
        wid = lax.axis_index("s") * _NC + lax.axis_index("c")
        base = wid * _BPW
        pltpu.sync_copy(idx_hbm.at[pl.ds(base, _BPW)], idx_v)
        pltpu.async_copy(table_hbm.at[idx_v], rows_v, sem).wait()
        pltpu.sync_copy(rows_v, out_hbm.at[pl.ds(base, _BPW)])

    return _sc_gather


def kernel(latents, embedding_weight):
    b, d, h, w = latents.shape
    zt3 = latents.reshape(b, d, h * w)
    idx, loss, perp = _tc_call(zt3, embedding_weight)
    emb_pad = jnp.pad(embedding_weight, ((0, 0), (0, _DPAD - d)))
    qflat = _sc_gather_call()(emb_pad, idx)[:, :d]
    quantized = jnp.transpose(qflat.reshape(b, h, w, d), (0, 3, 1, 2))
    return quantized, loss[0, 0], perp[0, 0]
